# baseline (speedup 1.0000x reference)
import functools

import jax
import jax.numpy as jnp
from jax.experimental import pallas as pl


_BT = 2048


def _gate_kernel(x_ref, wt_ref, b_ref, u_ref, o_ref):
    scores = jnp.dot(x_ref[...], wt_ref[...], preferred_element_type=jnp.float32)
    gumbels = -jnp.log(-jnp.log(u_ref[...]))
    y = scores + b_ref[...] + gumbels
    idx = jnp.argmax(y, axis=-1)
    expert = jax.lax.broadcasted_iota(jnp.int32, y.shape, 1)
    y_hard = (expert == idx[:, None]).astype(jnp.float32)
    m = jnp.max(y, axis=-1, keepdims=True)
    e = jnp.exp(y - m)
    y_soft = e / jnp.sum(e, axis=-1, keepdims=True)
    o_ref[...] = y_hard + y_soft - y_soft


@functools.partial(jax.jit, static_argnames=())
def kernel(x, gate_weight, gate_bias):
    n_tokens, d_model = x.shape
    n_experts = gate_weight.shape[0]
    u = jax.random.uniform(
        jax.random.key(42), (n_tokens, n_experts), dtype=x.dtype,
        minval=1e-20, maxval=1.0)
    wt = gate_weight.T
    b2 = gate_bias.reshape(1, n_experts)
    grid = (n_tokens // _BT,)
    return pl.pallas_call(
        _gate_kernel,
        grid=grid,
        in_specs=[
            pl.BlockSpec((_BT, d_model), lambda i: (i, 0)),
            pl.BlockSpec((d_model, n_experts), lambda i: (0, 0)),
            pl.BlockSpec((1, n_experts), lambda i: (0, 0)),
            pl.BlockSpec((_BT, n_experts), lambda i: (i, 0)),
        ],
        out_specs=pl.BlockSpec((_BT, n_experts), lambda i: (i, 0)),
        out_shape=jax.ShapeDtypeStruct((n_tokens, n_experts), x.dtype),
    )(x, wt, b2, u)


# speedup vs baseline: 1.0032x; 1.0032x over previous
import functools

import jax
import jax.numpy as jnp
from jax.experimental import pallas as pl
from jax.experimental.pallas import tpu as pltpu


_BT = 2048


def _gate_kernel(x_ref, wt_ref, b_ref, u_ref, o_ref):
    scores = jnp.dot(x_ref[...], wt_ref[...], preferred_element_type=jnp.float32)
    gumbels = -jnp.log(-jnp.log(u_ref[...]))
    y = scores + b_ref[...] + gumbels
    idx = jnp.argmax(y, axis=-1)
    expert = jax.lax.broadcasted_iota(jnp.int32, y.shape, 1)
    y_hard = (expert == idx[:, None]).astype(jnp.float32)
    m = jnp.max(y, axis=-1, keepdims=True)
    e = jnp.exp(y - m)
    y_soft = e / jnp.sum(e, axis=-1, keepdims=True)
    o_ref[...] = y_hard + y_soft - y_soft


@functools.partial(jax.jit, static_argnames=())
def kernel(x, gate_weight, gate_bias):
    n_tokens, d_model = x.shape
    n_experts = gate_weight.shape[0]
    u = jax.random.uniform(
        jax.random.key(42), (n_tokens, n_experts), dtype=x.dtype,
        minval=1e-20, maxval=1.0)
    wt = gate_weight.T
    b2 = gate_bias.reshape(1, n_experts)
    grid = (n_tokens // _BT,)
    return pl.pallas_call(
        _gate_kernel,
        grid=grid,
        in_specs=[
            pl.BlockSpec((_BT, d_model), lambda i: (i, 0)),
            pl.BlockSpec((d_model, n_experts), lambda i: (0, 0)),
            pl.BlockSpec((1, n_experts), lambda i: (0, 0)),
            pl.BlockSpec((_BT, n_experts), lambda i: (i, 0)),
        ],
        out_specs=pl.BlockSpec((_BT, n_experts), lambda i: (i, 0)),
        out_shape=jax.ShapeDtypeStruct((n_tokens, n_experts), x.dtype),
        compiler_params=pltpu.CompilerParams(
            dimension_semantics=("parallel",)),
    )(x, wt, b2, u)


# speedup vs baseline: 1.0293x; 1.0261x over previous
import functools

import jax
import jax.numpy as jnp
from jax.experimental import pallas as pl
from jax.experimental.pallas import tpu as pltpu


_BT = 4096


def _gate_kernel(x_ref, wt_ref, b_ref, u_ref, o_ref):
    scores = jnp.dot(x_ref[...], wt_ref[...], preferred_element_type=jnp.float32)
    gumbels = -jnp.log(-jnp.log(u_ref[...]))
    y = scores + b_ref[...] + gumbels
    idx = jnp.argmax(y, axis=-1)
    expert = jax.lax.broadcasted_iota(jnp.int32, y.shape, 1)
    y_hard = (expert == idx[:, None]).astype(jnp.float32)
    m = jnp.max(y, axis=-1, keepdims=True)
    e = jnp.exp(y - m)
    y_soft = e / jnp.sum(e, axis=-1, keepdims=True)
    o_ref[...] = y_hard + y_soft - y_soft


@functools.partial(jax.jit, static_argnames=())
def kernel(x, gate_weight, gate_bias):
    n_tokens, d_model = x.shape
    n_experts = gate_weight.shape[0]
    u = jax.random.uniform(
        jax.random.key(42), (n_tokens, n_experts), dtype=x.dtype,
        minval=1e-20, maxval=1.0)
    wt = gate_weight.T
    b2 = gate_bias.reshape(1, n_experts)
    grid = (n_tokens // _BT,)
    return pl.pallas_call(
        _gate_kernel,
        grid=grid,
        in_specs=[
            pl.BlockSpec((_BT, d_model), lambda i: (i, 0)),
            pl.BlockSpec((d_model, n_experts), lambda i: (0, 0)),
            pl.BlockSpec((1, n_experts), lambda i: (0, 0)),
            pl.BlockSpec((_BT, n_experts), lambda i: (i, 0)),
        ],
        out_specs=pl.BlockSpec((_BT, n_experts), lambda i: (i, 0)),
        out_shape=jax.ShapeDtypeStruct((n_tokens, n_experts), x.dtype),
        compiler_params=pltpu.CompilerParams(
            dimension_semantics=("parallel",)),
    )(x, wt, b2, u)
